# baseline (speedup 1.0000x reference)
import functools

import jax
import jax.numpy as jnp
from jax import lax
from jax.experimental import pallas as pl
from jax.experimental.pallas import tpu as pltpu
from jax.experimental.pallas import tpu_sc as plsc

N = 10000
E = 320000
D = 128
N_STEPS = 2
N_ETYPES = 4


_BN = 1000


def _msg_body(h_ref, w_ref, b_ref, out_ref):
    out_ref[0] = (
        jnp.dot(h_ref[...], w_ref[0], preferred_element_type=jnp.float32)
        + b_ref[0]
    )


def _msg_table(h, ws_t, bs):
    return pl.pallas_call(
        _msg_body,
        grid=(N_ETYPES, N // _BN),
        in_specs=[
            pl.BlockSpec((_BN, D), lambda e, nb: (nb, 0)),
            pl.BlockSpec((1, D, D), lambda e, nb: (e, 0, 0)),
            pl.BlockSpec((1, 1, D), lambda e, nb: (e, 0, 0)),
        ],
        out_specs=pl.BlockSpec((1, _BN, D), lambda e, nb: (e, nb, 0)),
        out_shape=jax.ShapeDtypeStruct((N_ETYPES, N, D), jnp.float32),
    )(h, ws_t, bs)


def _gru_body(p_ref, h_ref, wih_ref, whh_ref, bih_ref, bhh_ref, out_ref):
    a = p_ref[0] + p_ref[1]
    h = h_ref[...]
    gi = jnp.dot(a, wih_ref[...], preferred_element_type=jnp.float32) + bih_ref[0][None, :]
    gh = jnp.dot(h, whh_ref[...], preferred_element_type=jnp.float32) + bhh_ref[0][None, :]
    r = jax.nn.sigmoid(gi[:, :D] + gh[:, :D])
    z = jax.nn.sigmoid(gi[:, D : 2 * D] + gh[:, D : 2 * D])
    n = jnp.tanh(gi[:, 2 * D :] + r * gh[:, 2 * D :])
    out_ref[...] = (1.0 - z) * n + z * h


def _gru(parts, h, wih_t, whh_t, bih, bhh):
    return pl.pallas_call(
        _gru_body,
        grid=(N // _BN,),
        in_specs=[
            pl.BlockSpec((2, _BN, D), lambda nb: (0, nb, 0)),
            pl.BlockSpec((_BN, D), lambda nb: (nb, 0)),
            pl.BlockSpec((D, 3 * D), lambda nb: (0, 0)),
            pl.BlockSpec((D, 3 * D), lambda nb: (0, 0)),
            pl.BlockSpec((1, 3 * D), lambda nb: (0, 0)),
            pl.BlockSpec((1, 3 * D), lambda nb: (0, 0)),
        ],
        out_specs=pl.BlockSpec((_BN, D), lambda nb: (nb, 0)),
        out_shape=jax.ShapeDtypeStruct((N, D), jnp.float32),
    )(parts, h, wih_t, whh_t, bih, bhh)


def _gru_msg_body(p_ref, h_ref, wih_ref, whh_ref, bih_ref, bhh_ref,
                  wst_ref, bs_ref, hout_ref, tout_ref):
    a = p_ref[0] + p_ref[1]
    h = h_ref[...]
    gi = jnp.dot(a, wih_ref[...], preferred_element_type=jnp.float32) + bih_ref[0][None, :]
    gh = jnp.dot(h, whh_ref[...], preferred_element_type=jnp.float32) + bhh_ref[0][None, :]
    r = jax.nn.sigmoid(gi[:, :D] + gh[:, :D])
    z = jax.nn.sigmoid(gi[:, D : 2 * D] + gh[:, D : 2 * D])
    n = jnp.tanh(gi[:, 2 * D :] + r * gh[:, 2 * D :])
    hn = (1.0 - z) * n + z * h
    hout_ref[...] = hn
    for e in range(N_ETYPES):
        tout_ref[e] = (
            jnp.dot(hn, wst_ref[e], preferred_element_type=jnp.float32)
            + bs_ref[e]
        )


def _gru_msg(parts, h, wih_t, whh_t, bih, bhh, ws_t, bs):
    return pl.pallas_call(
        _gru_msg_body,
        grid=(N // _BN,),
        in_specs=[
            pl.BlockSpec((2, _BN, D), lambda nb: (0, nb, 0)),
            pl.BlockSpec((_BN, D), lambda nb: (nb, 0)),
            pl.BlockSpec((D, 3 * D), lambda nb: (0, 0)),
            pl.BlockSpec((D, 3 * D), lambda nb: (0, 0)),
            pl.BlockSpec((1, 3 * D), lambda nb: (0, 0)),
            pl.BlockSpec((1, 3 * D), lambda nb: (0, 0)),
            pl.BlockSpec((N_ETYPES, D, D), lambda nb: (0, 0, 0)),
            pl.BlockSpec((N_ETYPES, 1, D), lambda nb: (0, 0, 0)),
        ],
        out_specs=[
            pl.BlockSpec((_BN, D), lambda nb: (nb, 0)),
            pl.BlockSpec((N_ETYPES, _BN, D), lambda nb: (0, nb, 0)),
        ],
        out_shape=[
            jax.ShapeDtypeStruct((N, D), jnp.float32),
            jax.ShapeDtypeStruct((N_ETYPES, N, D), jnp.float32),
        ],
    )(parts, h, wih_t, whh_t, bih, bhh, ws_t, bs)


_NC = 2
_NS = 16
_NW = _NC * _NS
_EPW = E // _NW
_C = 40
_NCH = _EPW // _C
_NPAD = 10112
_RPS = _NPAD // _NS


_NB = 3
_NFULL = (_NCH // _NB) * _NB
_NTAIL = _NCH - _NFULL


@functools.cache
def _sc_aggregate_kernel():
    mesh = plsc.VectorSubcoreMesh(core_axis_name="c", subcore_axis_name="s")
    scratch = [
        pltpu.VMEM((_EPW,), jnp.int32),
        pltpu.VMEM_SHARED((_NPAD, D), jnp.float32),
    ]
    scratch += [pltpu.VMEM((1, _C), jnp.int32) for _ in range(_NB)]
    scratch += [pltpu.VMEM((_C, D), jnp.float32) for _ in range(_NB)]
    scratch += [pltpu.SemaphoreType.DMA for _ in range(3 * _NB)]
    return pl.kernel(
        _sc_aggregate_body,
        out_type=jax.ShapeDtypeStruct((_NC, _NS, _RPS, D), jnp.float32),
        mesh=mesh,
        scratch_types=scratch,
    )


def _sc_aggregate_body(whall, gidx, dst, zrows, out, idx_v, acc, *rest):
    dbuf = rest[:_NB]
    rows = rest[_NB : 2 * _NB]
    gsem = rest[2 * _NB : 3 * _NB]
    ssem = rest[3 * _NB : 4 * _NB]
    dsem = rest[4 * _NB : 5 * _NB]
    c = lax.axis_index("c")
    s = lax.axis_index("s")
    wid = c * _NS + s

    pltpu.sync_copy(gidx.at[pl.ds(wid * _EPW, _EPW)], idx_v)

    row0 = s * _RPS
    pltpu.sync_copy(zrows, acc.at[pl.ds(row0, _RPS)])
    plsc.subcore_barrier()

    def _gwait(b):
        pltpu.make_async_copy(whall.at[pl.ds(0, _C)], rows[b], gsem[b]).wait()

    def _swait(b):
        pltpu.make_async_copy(whall.at[pl.ds(0, _C)], rows[b], ssem[b]).wait()

    def _dwait(b):
        pltpu.make_async_copy(dst.at[0], dbuf[b], dsem[b]).wait()

    cbase = wid * _NCH
    for b in range(_NB):
        pltpu.async_copy(dst.at[cbase + b], dbuf[b], dsem[b])
        pltpu.async_copy(
            whall.at[idx_v.at[pl.ds(b * _C, _C)]], rows[b], gsem[b]
        )

    def _ring(g, _):
        t0 = g * _NB
        for b in range(_NB):
            _gwait(b)
            _dwait(b)
            pltpu.async_copy(rows[b], acc.at[dbuf[b].at[0]], ssem[b], add=True)
        for b in range(_NB):
            t = t0 + b
            _swait(b)

            @pl.when(t + _NB < _NCH)
            def _():
                pltpu.async_copy(
                    whall.at[idx_v.at[pl.ds((t + _NB) * _C, _C)]], rows[b], gsem[b]
                )
                pltpu.async_copy(dst.at[cbase + t + _NB], dbuf[b], dsem[b])

        return 0

    lax.fori_loop(0, _NCH // _NB, _ring, 0)

    for b in range(_NTAIL):
        _gwait(b)
        _dwait(b)
        pltpu.async_copy(rows[b], acc.at[dbuf[b].at[0]], ssem[b], add=True)
    for b in range(_NTAIL):
        _swait(b)
    plsc.subcore_barrier()

    pltpu.sync_copy(acc.at[pl.ds(row0, _RPS)], out.at[c, s])


_WHALL_SHAPE = (N_ETYPES * N, D)
_GIDX_SHAPE = (E,)
_DST_SHAPE = (_NW * _NCH, 1, _C)
_ZROWS_SHAPE = (_RPS, D)


def kernel(x, edge_index, etypes, Ws, bs, W_ih, W_hh, b_ih, b_hh):
    src = edge_index[0].astype(jnp.int32)
    dst = edge_index[1].astype(jnp.int32).reshape(_DST_SHAPE)
    gidx = etypes.astype(jnp.int32) * N + src
    ws_t = jnp.swapaxes(Ws, 1, 2)
    wih_t = W_ih.T
    whh_t = W_hh.T
    bs3 = bs.reshape(N_ETYPES, 1, D)
    bih = b_ih.reshape(1, 3 * D)
    bhh = b_hh.reshape(1, 3 * D)
    zrows = jnp.zeros(_ZROWS_SHAPE, jnp.float32)

    whall = _msg_table(x, ws_t, bs3).reshape(N_ETYPES * N, D)
    h = x
    for step in range(N_STEPS):
        parts = _sc_aggregate_kernel()(whall, gidx, dst, zrows)
        parts = parts.reshape(_NC, _NPAD, D)
        if step < N_STEPS - 1:
            h, whall = _gru_msg(parts, h, wih_t, whh_t, bih, bhh, ws_t, bs3)
            whall = whall.reshape(N_ETYPES * N, D)
        else:
            h = _gru(parts, h, wih_t, whh_t, bih, bhh)
    return h


# speedup vs baseline: 1.1702x; 1.1702x over previous
import functools

import jax
import jax.numpy as jnp
from jax import lax
from jax.experimental import pallas as pl
from jax.experimental.pallas import tpu as pltpu
from jax.experimental.pallas import tpu_sc as plsc

N = 10000
E = 320000
D = 128
N_STEPS = 2
N_ETYPES = 4


_BN = 1000


def _msg_body(h_ref, w_ref, b_ref, out_ref):
    h = h_ref[...]
    for e in range(N_ETYPES):
        out_ref[e] = (
            jnp.dot(h, w_ref[e], preferred_element_type=jnp.float32)
            + b_ref[e]
        )


def _msg_table(h, ws_t, bs):
    return pl.pallas_call(
        _msg_body,
        grid=(N // _BN,),
        in_specs=[
            pl.BlockSpec((_BN, D), lambda nb: (nb, 0)),
            pl.BlockSpec((N_ETYPES, D, D), lambda nb: (0, 0, 0)),
            pl.BlockSpec((N_ETYPES, 1, D), lambda nb: (0, 0, 0)),
        ],
        out_specs=pl.BlockSpec((N_ETYPES, _BN, D), lambda nb: (0, nb, 0)),
        out_shape=jax.ShapeDtypeStruct((N_ETYPES, N, D), jnp.float32),
    )(h, ws_t, bs)


def _gru_body(p_ref, h_ref, wih_ref, whh_ref, bih_ref, bhh_ref, out_ref):
    a = p_ref[0] + p_ref[1]
    h = h_ref[...]
    gi = jnp.dot(a, wih_ref[...], preferred_element_type=jnp.float32) + bih_ref[0][None, :]
    gh = jnp.dot(h, whh_ref[...], preferred_element_type=jnp.float32) + bhh_ref[0][None, :]
    r = jax.nn.sigmoid(gi[:, :D] + gh[:, :D])
    z = jax.nn.sigmoid(gi[:, D : 2 * D] + gh[:, D : 2 * D])
    n = jnp.tanh(gi[:, 2 * D :] + r * gh[:, 2 * D :])
    out_ref[...] = (1.0 - z) * n + z * h


def _gru(parts, h, wih_t, whh_t, bih, bhh):
    return pl.pallas_call(
        _gru_body,
        grid=(N // _BN,),
        in_specs=[
            pl.BlockSpec((2, _BN, D), lambda nb: (0, nb, 0)),
            pl.BlockSpec((_BN, D), lambda nb: (nb, 0)),
            pl.BlockSpec((D, 3 * D), lambda nb: (0, 0)),
            pl.BlockSpec((D, 3 * D), lambda nb: (0, 0)),
            pl.BlockSpec((1, 3 * D), lambda nb: (0, 0)),
            pl.BlockSpec((1, 3 * D), lambda nb: (0, 0)),
        ],
        out_specs=pl.BlockSpec((_BN, D), lambda nb: (nb, 0)),
        out_shape=jax.ShapeDtypeStruct((N, D), jnp.float32),
    )(parts, h, wih_t, whh_t, bih, bhh)


def _gru_msg_body(p_ref, h_ref, wih_ref, whh_ref, bih_ref, bhh_ref,
                  wst_ref, bs_ref, hout_ref, tout_ref):
    a = p_ref[0] + p_ref[1]
    h = h_ref[...]
    gi = jnp.dot(a, wih_ref[...], preferred_element_type=jnp.float32) + bih_ref[0][None, :]
    gh = jnp.dot(h, whh_ref[...], preferred_element_type=jnp.float32) + bhh_ref[0][None, :]
    r = jax.nn.sigmoid(gi[:, :D] + gh[:, :D])
    z = jax.nn.sigmoid(gi[:, D : 2 * D] + gh[:, D : 2 * D])
    n = jnp.tanh(gi[:, 2 * D :] + r * gh[:, 2 * D :])
    hn = (1.0 - z) * n + z * h
    hout_ref[...] = hn
    for e in range(N_ETYPES):
        tout_ref[e] = (
            jnp.dot(hn, wst_ref[e], preferred_element_type=jnp.float32)
            + bs_ref[e]
        )


def _gru_msg(parts, h, wih_t, whh_t, bih, bhh, ws_t, bs):
    return pl.pallas_call(
        _gru_msg_body,
        grid=(N // _BN,),
        in_specs=[
            pl.BlockSpec((2, _BN, D), lambda nb: (0, nb, 0)),
            pl.BlockSpec((_BN, D), lambda nb: (nb, 0)),
            pl.BlockSpec((D, 3 * D), lambda nb: (0, 0)),
            pl.BlockSpec((D, 3 * D), lambda nb: (0, 0)),
            pl.BlockSpec((1, 3 * D), lambda nb: (0, 0)),
            pl.BlockSpec((1, 3 * D), lambda nb: (0, 0)),
            pl.BlockSpec((N_ETYPES, D, D), lambda nb: (0, 0, 0)),
            pl.BlockSpec((N_ETYPES, 1, D), lambda nb: (0, 0, 0)),
        ],
        out_specs=[
            pl.BlockSpec((_BN, D), lambda nb: (nb, 0)),
            pl.BlockSpec((N_ETYPES, _BN, D), lambda nb: (0, nb, 0)),
        ],
        out_shape=[
            jax.ShapeDtypeStruct((N, D), jnp.float32),
            jax.ShapeDtypeStruct((N_ETYPES, N, D), jnp.float32),
        ],
    )(parts, h, wih_t, whh_t, bih, bhh, ws_t, bs)


_NC = 2
_NS = 16
_NW = _NC * _NS
_EPW = E // _NW
_C = 80
_NCH = _EPW // _C
_NPAD = 10112
_RPS = _NPAD // _NS


_NB = 3
_NFULL = (_NCH // _NB) * _NB
_NTAIL = _NCH - _NFULL


@functools.cache
def _sc_aggregate_kernel():
    mesh = plsc.VectorSubcoreMesh(core_axis_name="c", subcore_axis_name="s")
    scratch = [
        pltpu.VMEM((_EPW,), jnp.int32),
        pltpu.VMEM_SHARED((_NPAD, D), jnp.float32),
    ]
    scratch += [pltpu.VMEM((1, _C), jnp.int32) for _ in range(_NB)]
    scratch += [pltpu.VMEM((_C, D), jnp.float32) for _ in range(_NB)]
    scratch += [pltpu.SemaphoreType.DMA for _ in range(3 * _NB)]
    return pl.kernel(
        _sc_aggregate_body,
        out_type=jax.ShapeDtypeStruct((_NC, _NS, _RPS, D), jnp.float32),
        mesh=mesh,
        scratch_types=scratch,
    )


def _sc_aggregate_body(whall, gidx, dst, zrows, out, idx_v, acc, *rest):
    dbuf = rest[:_NB]
    rows = rest[_NB : 2 * _NB]
    gsem = rest[2 * _NB : 3 * _NB]
    ssem = rest[3 * _NB : 4 * _NB]
    dsem = rest[4 * _NB : 5 * _NB]
    c = lax.axis_index("c")
    s = lax.axis_index("s")
    wid = c * _NS + s

    pltpu.sync_copy(gidx.at[pl.ds(wid * _EPW, _EPW)], idx_v)

    row0 = s * _RPS
    pltpu.sync_copy(zrows, acc.at[pl.ds(row0, _RPS)])
    plsc.subcore_barrier()

    def _gwait(b):
        pltpu.make_async_copy(whall.at[pl.ds(0, _C)], rows[b], gsem[b]).wait()

    def _swait(b):
        pltpu.make_async_copy(whall.at[pl.ds(0, _C)], rows[b], ssem[b]).wait()

    def _dwait(b):
        pltpu.make_async_copy(dst.at[0], dbuf[b], dsem[b]).wait()

    cbase = wid * _NCH
    for b in range(_NB):
        pltpu.async_copy(dst.at[cbase + b], dbuf[b], dsem[b])
        pltpu.async_copy(
            whall.at[idx_v.at[pl.ds(b * _C, _C)]], rows[b], gsem[b]
        )

    def _ring(g, _):
        t0 = g * _NB
        for b in range(_NB):
            _gwait(b)
            _dwait(b)
            pltpu.async_copy(rows[b], acc.at[dbuf[b].at[0]], ssem[b], add=True)
        for b in range(_NB):
            t = t0 + b
            _swait(b)

            @pl.when(t + _NB < _NCH)
            def _():
                pltpu.async_copy(
                    whall.at[idx_v.at[pl.ds((t + _NB) * _C, _C)]], rows[b], gsem[b]
                )
                pltpu.async_copy(dst.at[cbase + t + _NB], dbuf[b], dsem[b])

        return 0

    lax.fori_loop(0, _NCH // _NB, _ring, 0)

    for b in range(_NTAIL):
        _gwait(b)
        _dwait(b)
        pltpu.async_copy(rows[b], acc.at[dbuf[b].at[0]], ssem[b], add=True)
    for b in range(_NTAIL):
        _swait(b)
    plsc.subcore_barrier()

    pltpu.sync_copy(acc.at[pl.ds(row0, _RPS)], out.at[c, s])


_WHALL_SHAPE = (N_ETYPES * N, D)
_GIDX_SHAPE = (E,)
_DST_SHAPE = (_NW * _NCH, 1, _C)
_ZROWS_SHAPE = (_RPS, D)


def kernel(x, edge_index, etypes, Ws, bs, W_ih, W_hh, b_ih, b_hh):
    src = edge_index[0].astype(jnp.int32)
    dst = edge_index[1].astype(jnp.int32).reshape(_DST_SHAPE)
    gidx = etypes.astype(jnp.int32) * N + src
    ws_t = jnp.swapaxes(Ws, 1, 2)
    wih_t = W_ih.T
    whh_t = W_hh.T
    bs3 = bs.reshape(N_ETYPES, 1, D)
    bih = b_ih.reshape(1, 3 * D)
    bhh = b_hh.reshape(1, 3 * D)
    zrows = jnp.zeros(_ZROWS_SHAPE, jnp.float32)

    whall = _msg_table(x, ws_t, bs3).reshape(N_ETYPES * N, D)
    h = x
    for step in range(N_STEPS):
        parts = _sc_aggregate_kernel()(whall, gidx, dst, zrows)
        parts = parts.reshape(_NC, _NPAD, D)
        if step < N_STEPS - 1:
            h, whall = _gru_msg(parts, h, wih_t, whh_t, bih, bhh, ws_t, bs3)
            whall = whall.reshape(N_ETYPES * N, D)
        else:
            h = _gru(parts, h, wih_t, whh_t, bih, bhh)
    return h


# speedup vs baseline: 1.2014x; 1.0266x over previous
import functools

import jax
import jax.numpy as jnp
from jax import lax
from jax.experimental import pallas as pl
from jax.experimental.pallas import tpu as pltpu
from jax.experimental.pallas import tpu_sc as plsc

N = 10000
E = 320000
D = 128
N_STEPS = 2
N_ETYPES = 4


_BN = 2000


def _msg_body(h_ref, w_ref, b_ref, out_ref):
    h = h_ref[...]
    for e in range(N_ETYPES):
        out_ref[e] = (
            jnp.dot(h, w_ref[e], preferred_element_type=jnp.float32)
            + b_ref[e]
        )


def _msg_table(h, ws_t, bs):
    return pl.pallas_call(
        _msg_body,
        grid=(N // _BN,),
        in_specs=[
            pl.BlockSpec((_BN, D), lambda nb: (nb, 0)),
            pl.BlockSpec((N_ETYPES, D, D), lambda nb: (0, 0, 0)),
            pl.BlockSpec((N_ETYPES, 1, D), lambda nb: (0, 0, 0)),
        ],
        out_specs=pl.BlockSpec((N_ETYPES, _BN, D), lambda nb: (0, nb, 0)),
        out_shape=jax.ShapeDtypeStruct((N_ETYPES, N, D), jnp.float32),
    )(h, ws_t, bs)


def _gru_body(p_ref, h_ref, wih_ref, whh_ref, bih_ref, bhh_ref, out_ref):
    a = p_ref[0] + p_ref[1]
    h = h_ref[...]
    gi = jnp.dot(a, wih_ref[...], preferred_element_type=jnp.float32) + bih_ref[0][None, :]
    gh = jnp.dot(h, whh_ref[...], preferred_element_type=jnp.float32) + bhh_ref[0][None, :]
    r = jax.nn.sigmoid(gi[:, :D] + gh[:, :D])
    z = jax.nn.sigmoid(gi[:, D : 2 * D] + gh[:, D : 2 * D])
    n = jnp.tanh(gi[:, 2 * D :] + r * gh[:, 2 * D :])
    out_ref[...] = (1.0 - z) * n + z * h


def _gru(parts, h, wih_t, whh_t, bih, bhh):
    return pl.pallas_call(
        _gru_body,
        grid=(N // _BN,),
        in_specs=[
            pl.BlockSpec((2, _BN, D), lambda nb: (0, nb, 0)),
            pl.BlockSpec((_BN, D), lambda nb: (nb, 0)),
            pl.BlockSpec((D, 3 * D), lambda nb: (0, 0)),
            pl.BlockSpec((D, 3 * D), lambda nb: (0, 0)),
            pl.BlockSpec((1, 3 * D), lambda nb: (0, 0)),
            pl.BlockSpec((1, 3 * D), lambda nb: (0, 0)),
        ],
        out_specs=pl.BlockSpec((_BN, D), lambda nb: (nb, 0)),
        out_shape=jax.ShapeDtypeStruct((N, D), jnp.float32),
    )(parts, h, wih_t, whh_t, bih, bhh)


def _gru_msg_body(p_ref, h_ref, wih_ref, whh_ref, bih_ref, bhh_ref,
                  wst_ref, bs_ref, hout_ref, tout_ref):
    a = p_ref[0] + p_ref[1]
    h = h_ref[...]
    gi = jnp.dot(a, wih_ref[...], preferred_element_type=jnp.float32) + bih_ref[0][None, :]
    gh = jnp.dot(h, whh_ref[...], preferred_element_type=jnp.float32) + bhh_ref[0][None, :]
    r = jax.nn.sigmoid(gi[:, :D] + gh[:, :D])
    z = jax.nn.sigmoid(gi[:, D : 2 * D] + gh[:, D : 2 * D])
    n = jnp.tanh(gi[:, 2 * D :] + r * gh[:, 2 * D :])
    hn = (1.0 - z) * n + z * h
    hout_ref[...] = hn
    for e in range(N_ETYPES):
        tout_ref[e] = (
            jnp.dot(hn, wst_ref[e], preferred_element_type=jnp.float32)
            + bs_ref[e]
        )


def _gru_msg(parts, h, wih_t, whh_t, bih, bhh, ws_t, bs):
    return pl.pallas_call(
        _gru_msg_body,
        grid=(N // _BN,),
        in_specs=[
            pl.BlockSpec((2, _BN, D), lambda nb: (0, nb, 0)),
            pl.BlockSpec((_BN, D), lambda nb: (nb, 0)),
            pl.BlockSpec((D, 3 * D), lambda nb: (0, 0)),
            pl.BlockSpec((D, 3 * D), lambda nb: (0, 0)),
            pl.BlockSpec((1, 3 * D), lambda nb: (0, 0)),
            pl.BlockSpec((1, 3 * D), lambda nb: (0, 0)),
            pl.BlockSpec((N_ETYPES, D, D), lambda nb: (0, 0, 0)),
            pl.BlockSpec((N_ETYPES, 1, D), lambda nb: (0, 0, 0)),
        ],
        out_specs=[
            pl.BlockSpec((_BN, D), lambda nb: (nb, 0)),
            pl.BlockSpec((N_ETYPES, _BN, D), lambda nb: (0, nb, 0)),
        ],
        out_shape=[
            jax.ShapeDtypeStruct((N, D), jnp.float32),
            jax.ShapeDtypeStruct((N_ETYPES, N, D), jnp.float32),
        ],
    )(parts, h, wih_t, whh_t, bih, bhh, ws_t, bs)


_NC = 2
_NS = 16
_NW = _NC * _NS
_EPW = E // _NW
_C = 80
_NCH = _EPW // _C
_NPAD = 10112
_RPS = _NPAD // _NS


_NB = 3
_NFULL = (_NCH // _NB) * _NB
_NTAIL = _NCH - _NFULL


@functools.cache
def _sc_aggregate_kernel():
    mesh = plsc.VectorSubcoreMesh(core_axis_name="c", subcore_axis_name="s")
    scratch = [
        pltpu.VMEM((_EPW,), jnp.int32),
        pltpu.VMEM_SHARED((_NPAD, D), jnp.float32),
    ]
    scratch += [pltpu.VMEM((1, _C), jnp.int32) for _ in range(_NB)]
    scratch += [pltpu.VMEM((_C, D), jnp.float32) for _ in range(_NB)]
    scratch += [pltpu.SemaphoreType.DMA for _ in range(3 * _NB)]
    return pl.kernel(
        _sc_aggregate_body,
        out_type=jax.ShapeDtypeStruct((_NC, _NS, _RPS, D), jnp.float32),
        mesh=mesh,
        scratch_types=scratch,
    )


def _sc_aggregate_body(whall, gidx, dst, zrows, out, idx_v, acc, *rest):
    dbuf = rest[:_NB]
    rows = rest[_NB : 2 * _NB]
    gsem = rest[2 * _NB : 3 * _NB]
    ssem = rest[3 * _NB : 4 * _NB]
    dsem = rest[4 * _NB : 5 * _NB]
    c = lax.axis_index("c")
    s = lax.axis_index("s")
    wid = c * _NS + s

    pltpu.sync_copy(gidx.at[pl.ds(wid * _EPW, _EPW)], idx_v)

    row0 = s * _RPS
    pltpu.sync_copy(zrows, acc.at[pl.ds(row0, _RPS)])
    plsc.subcore_barrier()

    def _gwait(b):
        pltpu.make_async_copy(whall.at[pl.ds(0, _C)], rows[b], gsem[b]).wait()

    def _swait(b):
        pltpu.make_async_copy(whall.at[pl.ds(0, _C)], rows[b], ssem[b]).wait()

    def _dwait(b):
        pltpu.make_async_copy(dst.at[0], dbuf[b], dsem[b]).wait()

    cbase = wid * _NCH
    for b in range(_NB):
        pltpu.async_copy(dst.at[cbase + b], dbuf[b], dsem[b])
        pltpu.async_copy(
            whall.at[idx_v.at[pl.ds(b * _C, _C)]], rows[b], gsem[b]
        )

    def _ring(g, _):
        t0 = g * _NB
        for b in range(_NB):
            _gwait(b)
            _dwait(b)
            pltpu.async_copy(rows[b], acc.at[dbuf[b].at[0]], ssem[b], add=True)
        for b in range(_NB):
            t = t0 + b
            _swait(b)

            @pl.when(t + _NB < _NCH)
            def _():
                pltpu.async_copy(
                    whall.at[idx_v.at[pl.ds((t + _NB) * _C, _C)]], rows[b], gsem[b]
                )
                pltpu.async_copy(dst.at[cbase + t + _NB], dbuf[b], dsem[b])

        return 0

    lax.fori_loop(0, _NCH // _NB, _ring, 0)

    for b in range(_NTAIL):
        _gwait(b)
        _dwait(b)
        pltpu.async_copy(rows[b], acc.at[dbuf[b].at[0]], ssem[b], add=True)
    for b in range(_NTAIL):
        _swait(b)
    plsc.subcore_barrier()

    pltpu.sync_copy(acc.at[pl.ds(row0, _RPS)], out.at[c, s])


_WHALL_SHAPE = (N_ETYPES * N, D)
_GIDX_SHAPE = (E,)
_DST_SHAPE = (_NW * _NCH, 1, _C)
_ZROWS_SHAPE = (_RPS, D)


def kernel(x, edge_index, etypes, Ws, bs, W_ih, W_hh, b_ih, b_hh):
    src = edge_index[0].astype(jnp.int32)
    dst = edge_index[1].astype(jnp.int32).reshape(_DST_SHAPE)
    gidx = etypes.astype(jnp.int32) * N + src
    ws_t = jnp.swapaxes(Ws, 1, 2)
    wih_t = W_ih.T
    whh_t = W_hh.T
    bs3 = bs.reshape(N_ETYPES, 1, D)
    bih = b_ih.reshape(1, 3 * D)
    bhh = b_hh.reshape(1, 3 * D)
    zrows = jnp.zeros(_ZROWS_SHAPE, jnp.float32)

    whall = _msg_table(x, ws_t, bs3).reshape(N_ETYPES * N, D)
    h = x
    for step in range(N_STEPS):
        parts = _sc_aggregate_kernel()(whall, gidx, dst, zrows)
        parts = parts.reshape(_NC, _NPAD, D)
        if step < N_STEPS - 1:
            h, whall = _gru_msg(parts, h, wih_t, whh_t, bih, bhh, ws_t, bs3)
            whall = whall.reshape(N_ETYPES * N, D)
        else:
            h = _gru(parts, h, wih_t, whh_t, bih, bhh)
    return h
